# baseline (speedup 1.0000x reference)
import functools

import jax
import jax.numpy as jnp
from jax import lax
from jax.experimental import pallas as pl
from jax.experimental.pallas import tpu as pltpu
from jax.experimental.pallas import tpu_sc as plsc

N_USERS = 5000
N_ITEMS = 5000
N = N_USERS + N_ITEMS
E = 320000
D = 128
LEAKY = 0.5

NC = 2
NS = 16
NW = NC * NS
L = 16

NP = 10240
EPT = 10240
EP = NW * EPT
K = 64
NSUB = EPT // K
RPT = N // NS
ZR = 25
DR = 16
ECH = 512
NB = K // L


def _bcast_lane(v16, lane):
    idx = jnp.full((L,), lane, dtype=jnp.int32)
    return v16.at[idx].get(mode="promise_in_bounds")


_sc_mesh = plsc.VectorSubcoreMesh(core_axis_name="c", subcore_axis_name="s")


@functools.partial(
    pl.kernel,
    out_type=jax.ShapeDtypeStruct((NC, NP, D), jnp.float32),
    mesh=_sc_mesh,
    scratch_types=[
        pltpu.VMEM((EPT,), jnp.int32),
        pltpu.VMEM((EPT,), jnp.int32),
        pltpu.VMEM((EPT,), jnp.float32),
        pltpu.VMEM((2, K), jnp.int32),
        pltpu.VMEM((2, K, D), jnp.float32),
        pltpu.VMEM((ZR, D), jnp.float32),
        pltpu.VMEM((DR, D), jnp.float32),
        pltpu.VMEM_SHARED((N, D), jnp.float32),
        pltpu.SemaphoreType.DMA,
        pltpu.SemaphoreType.DMA,
    ],
    compiler_params=pltpu.CompilerParams(use_tc_tiling_on_sc=False),
)
def _spmm_partial(x_hbm, g_hbm, s_hbm, v_hbm, out_hbm,
                  gidx_v, sidx_v, vals_v, sidx2, rows2, zero_v, drain_v,
                  acc_sh, semg, sems):
    c = lax.axis_index("c")
    s = lax.axis_index("s")
    wid = s * NC + c

    def eload(q, _):
        sl = pl.ds(q * ECH, ECH)
        pltpu.sync_copy(g_hbm.at[wid, sl], gidx_v.at[sl])
        pltpu.sync_copy(s_hbm.at[wid, sl], sidx_v.at[sl])
        pltpu.sync_copy(v_hbm.at[wid, sl], vals_v.at[sl])
        return 0
    lax.fori_loop(0, EPT // ECH, eload, 0)

    def zrow(k, _):
        for r in range(D // L):
            zero_v[k, pl.ds(r * L, L)] = jnp.zeros((L,), jnp.float32)
        return 0
    lax.fori_loop(0, ZR, zrow, 0)
    def zacc(q, _):
        pltpu.sync_copy(zero_v, acc_sh.at[pl.ds(s * RPT + q * ZR, ZR)])
        return 0
    lax.fori_loop(0, RPT // ZR, zacc, 0)
    plsc.subcore_barrier()

    def drain(sem):
        for _ in range(K // DR):
            pltpu.make_async_copy(
                x_hbm.at[pl.ds(0, DR)], drain_v, sem).wait()

    def stage_scale(q, u):
        e0 = q * K
        for b in range(NB):
            sidx2[u, pl.ds(b * L, L)] = sidx_v[pl.ds(e0 + b * L, L)]

        def scale16(b, _):
            v16 = vals_v[pl.ds(e0 + b * L, L)]
            for l in range(L):
                bc = _bcast_lane(v16, l)
                k = b * L + l
                for r in range(D // L):
                    sl = pl.ds(r * L, L)
                    rows2[u, k, sl] = rows2[u, k, sl] * bc
            return 0
        lax.fori_loop(0, NB, scale16, 0)

    def body(i, _):
        par = jnp.bitwise_and(i, 1)
        prv = jnp.bitwise_and(i + 1, 1)

        @pl.when(i >= 2)
        def _():
            drain(sems)

        @pl.when(i < NSUB)
        def _():
            pltpu.async_copy(
                x_hbm.at[gidx_v.at[pl.ds(i * K, K)]], rows2.at[par], semg)

        @pl.when(i >= 1)
        def _():
            q = i - 1
            drain(semg)
            stage_scale(q, prv)
            pltpu.async_copy(
                rows2.at[prv], acc_sh.at[sidx2.at[prv]], sems, add=True)
        return 0
    lax.fori_loop(0, NSUB + 1, body, 0)
    drain(sems)

    plsc.subcore_barrier()

    for q in range(RPT // 125):
        off = s * RPT + q * 125
        pltpu.sync_copy(acc_sh.at[pl.ds(off, 125)],
                        out_hbm.at[c, pl.ds(off, 125)])


def _combine(p, leaky):
    def body(p_ref, o_ref):
        x = p_ref[0] + p_ref[1]
        if leaky:
            x = jnp.where(x >= 0, x, LEAKY * x)
        o_ref[...] = x

    rows = 1024
    return pl.pallas_call(
        body,
        out_shape=jax.ShapeDtypeStruct((NP, D), jnp.float32),
        grid=(NP // rows,),
        in_specs=[pl.BlockSpec((2, rows, D), lambda i: (0, i, 0))],
        out_specs=pl.BlockSpec((rows, D), lambda i: (i, 0)),
    )(p)


def kernel(user_emb, item_emb, edge_index, adj_vals):
    x = jnp.concatenate([
        user_emb, item_emb,
        jnp.zeros((NP - N, D), jnp.float32)], axis=0)
    pad = EP - E
    rows = jnp.concatenate(
        [edge_index[0], jnp.zeros((pad,), jnp.int32)]).reshape(NW, EPT)
    cols = jnp.concatenate(
        [edge_index[1], jnp.zeros((pad,), jnp.int32)]).reshape(NW, EPT)
    vals = jnp.concatenate(
        [adj_vals, jnp.zeros((pad,), jnp.float32)]).reshape(NW, EPT)

    for _ in range(2):
        p = _spmm_partial(x, rows, cols, vals)
        t = _combine(p, leaky=False)
        p = _spmm_partial(t, cols, rows, vals)
        x = _combine(p, leaky=True)

    return x[:N_USERS], x[N_USERS:N]


# speedup vs baseline: 1.2724x; 1.2724x over previous
import functools

import jax
import jax.numpy as jnp
from jax import lax
from jax.experimental import pallas as pl
from jax.experimental.pallas import tpu as pltpu
from jax.experimental.pallas import tpu_sc as plsc

N_USERS = 5000
N_ITEMS = 5000
N = N_USERS + N_ITEMS
E = 320000
D = 128
LEAKY = 0.5

NC = 2
NS = 16
NW = NC * NS
L = 16

NP = 10240
EPT = 10240
EP = NW * EPT
K = 128
NSUB = EPT // K
NPAIR = NSUB // 2
RPT = N // NS
ZR = 125
NB = K // L

EBYTES = 3 * K * 4
RBYTES = K * D * 4


def _bcast_lane(v16, lane):
    idx = jnp.full((L,), lane, dtype=jnp.int32)
    return v16.at[idx].get(mode="promise_in_bounds")


_sc_mesh = plsc.VectorSubcoreMesh(core_axis_name="c", subcore_axis_name="s")


@functools.partial(
    pl.kernel,
    out_type=jax.ShapeDtypeStruct((NC, NP, D), jnp.float32),
    mesh=_sc_mesh,
    scratch_types=[
        pltpu.VMEM((3, K), jnp.int32),
        pltpu.VMEM((3, K), jnp.int32),
        pltpu.VMEM((K,), jnp.int32),
        pltpu.VMEM((K,), jnp.int32),
        pltpu.VMEM((K, D), jnp.float32),
        pltpu.VMEM((K, D), jnp.float32),
        pltpu.VMEM((ZR, D), jnp.float32),
        pltpu.VMEM_SHARED((N, D), jnp.float32),
        pltpu.SemaphoreType.DMA,
        pltpu.SemaphoreType.DMA,
        pltpu.SemaphoreType.DMA,
    ],
    compiler_params=pltpu.CompilerParams(
        use_tc_tiling_on_sc=False, needs_layout_passes=False),
)
def _spmm_partial(x_hbm, ed_hbm, out_hbm,
                  eda, edb, sidxa, sidxb, rowsa, rowsb, zero_v, acc_sh,
                  seme, semg, sems):
    c = lax.axis_index("c")
    s = lax.axis_index("s")
    wid = s * NC + c

    def zrow(k, _):
        for r in range(D // L):
            zero_v[k, pl.ds(r * L, L)] = jnp.zeros((L,), jnp.float32)
        return 0
    lax.fori_loop(0, ZR, zrow, 0)
    def zacc(q, _):
        pltpu.sync_copy(zero_v, acc_sh.at[pl.ds(s * RPT + q * ZR, ZR)])
        return 0
    lax.fori_loop(0, RPT // ZR, zacc, 0)
    plsc.subcore_barrier()

    def estart(q, ebuf):
        pltpu.async_copy(ed_hbm.at[wid, q], ebuf, seme)

    def edrain():
        pltpu.make_async_copy(ed_hbm.at[wid, 0], eda, seme).wait()

    def gstart(q, ebuf, rows):
        pltpu.async_copy(x_hbm.at[ebuf.at[0]], rows, semg)

    def gdrain():
        pltpu.make_async_copy(x_hbm.at[pl.ds(0, K)], rowsa, semg).wait()

    def sstart(rows, sidx1):
        pltpu.async_copy(rows, acc_sh.at[sidx1], sems, add=True)

    def sdrain():
        pltpu.make_async_copy(rowsa, acc_sh.at[pl.ds(0, K)], sems).wait()

    def scale(ebuf, rows, sidx1):
        for b in range(NB):
            sidx1[pl.ds(b * L, L)] = ebuf[1, pl.ds(b * L, L)]
        for b in range(NB):
            v16 = plsc.bitcast(ebuf[2, pl.ds(b * L, L)], jnp.float32)
            for l in range(L):
                bc = _bcast_lane(v16, l)
                k = b * L + l
                for r in range(D // L):
                    sl = pl.ds(r * L, L)
                    rows[k, sl] = rows[k, sl] * bc

    estart(0, eda)
    estart(1, edb)
    edrain()
    gstart(0, eda, rowsa)

    def body(i, _):
        qa = 2 * i

        edrain()

        @pl.when(i >= 1)
        def _():
            sdrain()
        gstart(qa + 1, edb, rowsb)
        gdrain()
        scale(eda, rowsa, sidxa)

        @pl.when(i + 1 < NPAIR)
        def _():
            estart(qa + 2, eda)
        sstart(rowsa, sidxa)

        @pl.when(i + 1 < NPAIR)
        def _():
            edrain()
        sdrain()

        @pl.when(i + 1 < NPAIR)
        def _():
            gstart(qa + 2, eda, rowsa)
        gdrain()
        scale(edb, rowsb, sidxb)

        @pl.when(i + 1 < NPAIR)
        def _():
            estart(qa + 3, edb)
        sstart(rowsb, sidxb)
        return 0
    lax.fori_loop(0, NPAIR, body, 0)
    sdrain()

    plsc.subcore_barrier()

    for q in range(RPT // ZR):
        off = s * RPT + q * ZR
        pltpu.sync_copy(acc_sh.at[pl.ds(off, ZR)],
                        out_hbm.at[c, pl.ds(off, ZR)])


def _combine(p, leaky):
    def body(p_ref, o_ref):
        x = p_ref[0] + p_ref[1]
        if leaky:
            x = jnp.where(x >= 0, x, LEAKY * x)
        o_ref[...] = x

    rows = 1024
    return pl.pallas_call(
        body,
        out_shape=jax.ShapeDtypeStruct((NP, D), jnp.float32),
        grid=(NP // rows,),
        in_specs=[pl.BlockSpec((2, rows, D), lambda i: (0, i, 0))],
        out_specs=pl.BlockSpec((rows, D), lambda i: (i, 0)),
    )(p)


def kernel(user_emb, item_emb, edge_index, adj_vals):
    x = jnp.concatenate([
        user_emb, item_emb,
        jnp.zeros((NP - N, D), jnp.float32)], axis=0)
    pad = EP - E

    def chunked(a):
        return jnp.concatenate(
            [a, jnp.zeros((pad,), a.dtype)]).reshape(NW, NSUB, K)

    ri = chunked(edge_index[0])
    ci = chunked(edge_index[1])
    vi = chunked(jax.lax.bitcast_convert_type(adj_vals, jnp.int32))
    ed_fwd = jnp.stack([ri, ci, vi], axis=2)
    ed_bwd = jnp.stack([ci, ri, vi], axis=2)

    for _ in range(2):
        p = _spmm_partial(x, ed_fwd)
        t = _combine(p, leaky=False)
        p = _spmm_partial(t, ed_bwd)
        x = _combine(p, leaky=True)

    return x[:N_USERS], x[N_USERS:N]
